# baseline (speedup 1.0000x reference)
import functools

import jax
import jax.numpy as jnp
from jax import lax
from jax.experimental import pallas as pl
from jax.experimental.pallas import tpu as pltpu
from jax.experimental.pallas import tpu_sc as plsc

NC = 2
NS = 16
NW = NC * NS
CW = 128
H2 = 640
F = 128
CP = 16


def _mesh():
    return plsc.VectorSubcoreMesh(
        core_axis_name="c", subcore_axis_name="s", num_cores=NC, num_subcores=NS
    )


def _gather_call(feats, ctab, srcg, dstg, epad, nch, interpret=False):
    cpt = nch * CW
    f32 = jnp.float32

    @functools.partial(
        pl.kernel,
        out_type=[
            jax.ShapeDtypeStruct((epad, F), f32),
            jax.ShapeDtypeStruct((epad, F), f32),
            jax.ShapeDtypeStruct((epad, F), f32),
            jax.ShapeDtypeStruct((epad, F), f32),
        ],
        mesh=_mesh(),
        scratch_types=[
            pltpu.VMEM((nch, CW), jnp.int32),
            pltpu.VMEM((nch, CW), jnp.int32),
            pltpu.VMEM((2, CW, F), f32),
            pltpu.VMEM((2, CW, F), f32),
            pltpu.VMEM((2, CW, CP), f32),
            pltpu.VMEM((2, CW, CP), f32),
            pltpu.SemaphoreType.DMA,
            pltpu.SemaphoreType.DMA,
            pltpu.SemaphoreType.DMA,
            pltpu.SemaphoreType.DMA,
        ],
        compiler_params=pltpu.CompilerParams(use_tc_tiling_on_sc=False),
        interpret=interpret,
    )
    def k(feats_hbm, ctab_hbm, srcg_hbm, dstg_hbm, xi_hbm, xj_hbm, ci_hbm, cj_hbm,
          idxs_v, idxd_v, r_a, r_b, r_c, r_d, gs0, gs1, ws0, ws1):
        c = lax.axis_index("c")
        s = lax.axis_index("s")
        base = (c * NS + s) * cpt
        gs = (gs0, gs1)
        ws = (ws0, ws1)
        pltpu.sync_copy(srcg_hbm.at[c, s], idxs_v)
        pltpu.sync_copy(dstg_hbm.at[c, s], idxd_v)

        def fire_gather(j, b):
            pltpu.async_copy(feats_hbm.at[idxd_v.at[j]], r_a.at[b], gs[b])
            pltpu.async_copy(feats_hbm.at[idxs_v.at[j]], r_b.at[b], gs[b])
            pltpu.async_copy(ctab_hbm.at[idxd_v.at[j]], r_c.at[b], gs[b])
            pltpu.async_copy(ctab_hbm.at[idxs_v.at[j]], r_d.at[b], gs[b])

        def wait_gather(b):
            pltpu.make_async_copy(feats_hbm.at[idxd_v.at[0]], r_a.at[b], gs[b]).wait()
            pltpu.make_async_copy(feats_hbm.at[idxs_v.at[0]], r_b.at[b], gs[b]).wait()
            pltpu.make_async_copy(ctab_hbm.at[idxd_v.at[0]], r_c.at[b], gs[b]).wait()
            pltpu.make_async_copy(ctab_hbm.at[idxs_v.at[0]], r_d.at[b], gs[b]).wait()

        def fire_writes(j, b):
            off = base + j * CW
            pltpu.async_copy(r_a.at[b], xi_hbm.at[pl.ds(off, CW)], ws[b])
            pltpu.async_copy(r_b.at[b], xj_hbm.at[pl.ds(off, CW)], ws[b])
            pltpu.async_copy(r_c.at[b], ci_hbm.at[pl.ds(off, CW), pl.ds(0, CP)], ws[b])
            pltpu.async_copy(r_d.at[b], cj_hbm.at[pl.ds(off, CW), pl.ds(0, CP)], ws[b])

        def wait_writes(b):
            pltpu.make_async_copy(r_a.at[b], xi_hbm.at[pl.ds(0, CW)], ws[b]).wait()
            pltpu.make_async_copy(r_b.at[b], xj_hbm.at[pl.ds(0, CW)], ws[b]).wait()
            pltpu.make_async_copy(r_c.at[b], ci_hbm.at[pl.ds(0, CW), pl.ds(0, CP)], ws[b]).wait()
            pltpu.make_async_copy(r_d.at[b], cj_hbm.at[pl.ds(0, CW), pl.ds(0, CP)], ws[b]).wait()

        def body(g, carry):
            j0 = 2 * g

            @pl.when(g > 0)
            def _():
                wait_writes(0)

            fire_gather(j0, 0)
            wait_gather(0)
            fire_writes(j0, 0)

            @pl.when(g > 0)
            def _():
                wait_writes(1)

            fire_gather(j0 + 1, 1)
            wait_gather(1)
            fire_writes(j0 + 1, 1)
            return carry

        lax.fori_loop(0, nch // 2, body, 0)
        wait_writes(0)
        wait_writes(1)

    return k(feats, ctab, srcg, dstg)


def _scatter_call(eout, dsts, zer, n, nd, epad, nch, interpret=False):
    cpt = nch * CW
    zr = nd // NS
    npr = n // NS
    f32 = jnp.float32

    @functools.partial(
        pl.kernel,
        out_type=jax.ShapeDtypeStruct((NC, n, F), f32),
        mesh=_mesh(),
        scratch_types=[
            pltpu.VMEM((nch, CW), jnp.int32),
            pltpu.VMEM((max(CW, npr), 32), f32),
            pltpu.VMEM_SHARED((nd, 32), f32),
            pltpu.SemaphoreType.DMA,
        ],
        compiler_params=pltpu.CompilerParams(use_tc_tiling_on_sc=False),
        interpret=interpret,
    )
    def k(eout_hbm, dsts_hbm, zer_hbm, out_hbm, idx_v, rows_v, acc_sh, sem):
        c = lax.axis_index("c")
        s = lax.axis_index("s")
        base = (c * NS + s) * cpt
        pltpu.sync_copy(dsts_hbm.at[c, s], idx_v)
        pltpu.sync_copy(zer_hbm.at[pl.ds(s * zr, zr)], acc_sh.at[pl.ds(s * zr, zr)])
        plsc.subcore_barrier()

        def body(j, carry):
            cp = pltpu.async_copy(
                eout_hbm.at[pl.ds(base + j * CW, CW), pl.ds(0, 32)],
                rows_v.at[pl.ds(0, CW)], sem,
            )
            cp.wait()
            pltpu.sync_copy(rows_v.at[pl.ds(0, CW)], acc_sh.at[idx_v.at[j]], add=True)
            return carry

        lax.fori_loop(0, nch, body, 0)
        plsc.subcore_barrier()
        pltpu.sync_copy(acc_sh.at[pl.ds(s * npr, npr)], rows_v.at[pl.ds(0, npr)])
        pltpu.sync_copy(rows_v.at[pl.ds(0, npr)],
                        out_hbm.at[c, pl.ds(s * npr, npr), pl.ds(0, 32)])

    return k(eout, dsts, zer)


def _edge_body(xi, xj, ci, cj, ea, w1ij, w1a, w1d, b1, w2, b2, wc1, bc1, wc2,
               bc2, out):
    rel = cj[:, :CP] - ci[:, :CP]
    rd = jnp.sum(rel * rel, axis=1, keepdims=True)
    xij = jnp.concatenate([xi[...], xj[...]], axis=1).astype(jnp.bfloat16)
    h = jnp.dot(xij, w1ij[...], preferred_element_type=jnp.float32)
    h = h + jnp.dot(ea[...], w1a[...], preferred_element_type=jnp.float32)
    h = h + rd * w1d[...] + b1[...]
    h = h * jax.nn.sigmoid(h)
    m = jnp.dot(h.astype(jnp.bfloat16), w2[...],
                preferred_element_type=jnp.float32) + b2[...]
    m = m * jax.nn.sigmoid(m)
    q = jnp.dot(m, wc1[...], preferred_element_type=jnp.float32) + bc1[...]
    q = q * jax.nn.sigmoid(q)
    cw = jnp.sum(q * wc2[...], axis=1, keepdims=True) + bc2[0, 0]
    out[:, :32] = jnp.concatenate([m, cw * rel], axis=1)


def _edge_call(xi, xj, ci, cj, ea, w1ij, w1a, w1d, b1, w2, b2, wc1, bc1, wc2,
               bc2, epad, be, interpret=False):
    grid = (epad // be,)
    row = lambda i: (i, 0)
    full = lambda i: (0, 0)
    return pl.pallas_call(
        _edge_body,
        grid=grid,
        in_specs=[
            pl.BlockSpec((be, F), row),
            pl.BlockSpec((be, F), row),
            pl.BlockSpec((be, F), row),
            pl.BlockSpec((be, F), row),
            pl.BlockSpec((be, 16), row),
            pl.BlockSpec((2 * F, H2), full),
            pl.BlockSpec((16, H2), full),
            pl.BlockSpec((1, H2), full),
            pl.BlockSpec((1, H2), full),
            pl.BlockSpec((H2, 16), full),
            pl.BlockSpec((1, 16), full),
            pl.BlockSpec((16, 64), full),
            pl.BlockSpec((1, 64), full),
            pl.BlockSpec((1, 64), full),
            pl.BlockSpec((1, 1), full),
        ],
        out_specs=pl.BlockSpec((be, F), row),
        out_shape=jax.ShapeDtypeStruct((epad, F), jnp.float32),
        interpret=interpret,
    )(xi, xj, ci, cj, ea, w1ij, w1a, w1d, b1, w2, b2, wc1, bc1, wc2, bc2)


def _node_call(feats, aggs, ctab, w1f, w1m, b1, w2, b2, n, bn, interpret=False):
    na = len(aggs)

    def body(*refs):
        feats_r = refs[0]
        agg_rs = refs[1:1 + na]
        ctab_r, w1f_r, w1m_r, b1_r, w2_r, b2_r = refs[1 + na:1 + na + 6]
        nfeats, nctab = refs[-2:]
        a = agg_rs[0][...]
        for r in agg_rs[1:]:
            a = a + r[...]
        mi = a[:, :16]
        mh = a[:, 16:32]
        nctab[...] = ctab_r[...] + mh
        t = jnp.dot(feats_r[...], w1f_r[...], preferred_element_type=jnp.float32)
        t = t + jnp.dot(mi, w1m_r[...], preferred_element_type=jnp.float32) + b1_r[...]
        t = t * jax.nn.sigmoid(t)
        nfeats[...] = feats_r[...] + jnp.dot(t, w2_r[...], preferred_element_type=jnp.float32) + b2_r[...]

    grid = (n // bn,)
    row = lambda i: (i, 0)
    full = lambda i: (0, 0)
    return pl.pallas_call(
        body,
        grid=grid,
        in_specs=[pl.BlockSpec((bn, F), row)]
        + [pl.BlockSpec((bn, F), row)] * na
        + [
            pl.BlockSpec((bn, CP), row),
            pl.BlockSpec((F, 256), full),
            pl.BlockSpec((16, 256), full),
            pl.BlockSpec((1, 256), full),
            pl.BlockSpec((256, F), full),
            pl.BlockSpec((1, F), full),
        ],
        out_specs=[
            pl.BlockSpec((bn, F), row),
            pl.BlockSpec((bn, CP), row),
        ],
        out_shape=[
            jax.ShapeDtypeStruct((n, F), jnp.float32),
            jax.ShapeDtypeStruct((n, CP), jnp.float32),
        ],
        interpret=interpret,
    )(feats, *aggs, ctab, w1f, w1m, b1, w2, b2)


def _pad2(w, rows, cols):
    return jnp.pad(w, ((0, rows - w.shape[0]), (0, cols - w.shape[1])))


def kernel(x, edge_index, batch, edge_attr, We1, be1, We2, be2, Wc1, bc1, Wc2,
           bc2, Wn1, bn1, Wn2, bn2):
    n = x.shape[0]
    e = edge_index.shape[1]
    n_layers = We1.shape[0]
    nch = (e + NW * CW - 1) // (NW * CW)
    nch = ((nch + 3) // 4) * 4
    epad = nch * NW * CW
    if nch >= 40 and (nch - 8) % 6 == 0:
        splits = [8] + [(nch - 8) // 3] * 3
    else:
        splits = [nch // 2, nch // 2]
    nd = ((n + 1 + NS - 1) // NS) * NS
    bn = 1000 if n % 1000 == 0 else n
    be = 4096

    src = edge_index[0]
    dst = edge_index[1]
    pad = epad - e
    spread = jnp.arange(pad, dtype=jnp.int32) % n
    srcg = jnp.concatenate([src, spread]).reshape(NC, NS, nch, CW)
    dstg = jnp.concatenate([dst, spread]).reshape(NC, NS, nch, CW)
    dsts = jnp.pad(dst, (0, pad), constant_values=n).reshape(NC, NS, nch, CW)
    eap = jnp.pad(edge_attr, ((0, pad), (0, 0))).reshape(NW, nch, CW, 16)
    zer = jnp.zeros((nd, 32), jnp.float32)
    offs = [sum(splits[:i]) for i in range(len(splits))]
    srcg_h = [srcg[:, :, o:o + s, :] for o, s in zip(offs, splits)]
    dstg_h = [dstg[:, :, o:o + s, :] for o, s in zip(offs, splits)]
    dsts_h = [dsts[:, :, o:o + s, :] for o, s in zip(offs, splits)]
    eap_h = [eap[:, o:o + s].reshape(s * NW * CW, 16) for o, s in zip(offs, splits)]

    feats = x[:, 3:]
    ctab = jnp.pad(x[:, :3], ((0, 0), (0, CP - 3)))

    for l in range(n_layers):
        w1 = We1[l]
        w1ij = _pad2(w1[:2 * F], 2 * F, H2).astype(jnp.bfloat16)
        w1a = _pad2(w1[2 * F:2 * F + 16], 16, H2)
        w1d = _pad2(w1[2 * F + 16:], 1, H2)
        b1 = _pad2(be1[l][None, :], 1, H2)
        w2 = _pad2(We2[l], H2, 16).astype(jnp.bfloat16)
        b2 = be2[l][None, :]
        wc1 = Wc1[l]
        bc1l = bc1[l][None, :]
        wc2 = Wc2[l].reshape(1, 64)
        bc2l = bc2[l].reshape(1, 1)
        w1f = Wn1[l][:F]
        w1m = Wn1[l][F:]
        bn1l = bn1[l][None, :]
        wn2 = Wn2[l]
        bn2l = bn2[l][None, :]

        aggs = []
        for h, s in enumerate(splits):
            epad_s = s * NW * CW
            xi, xj, ci, cj = _gather_call(feats, ctab, srcg_h[h], dstg_h[h],
                                          epad_s, s)
            eout = _edge_call(xi, xj, ci, cj, eap_h[h], w1ij, w1a, w1d, b1,
                              w2, b2, wc1, bc1l, wc2, bc2l, epad_s, be)
            ah = _scatter_call(eout, dsts_h[h], zer, n, nd, epad_s, s)
            aggs.extend([ah[0], ah[1]])
        feats, ctab = _node_call(feats, aggs, ctab, w1f, w1m, bn1l, wn2, bn2l,
                                 n, bn)

    return jnp.concatenate([ctab[:, :3], feats], axis=1)


# speedup vs baseline: 1.0298x; 1.0298x over previous
import functools

import jax
import jax.numpy as jnp
from jax import lax
from jax.experimental import pallas as pl
from jax.experimental.pallas import tpu as pltpu
from jax.experimental.pallas import tpu_sc as plsc

NC = 2
NS = 16
NW = NC * NS
CW = 128
H2 = 640
F = 128
CP = 16


def _mesh():
    return plsc.VectorSubcoreMesh(
        core_axis_name="c", subcore_axis_name="s", num_cores=NC, num_subcores=NS
    )


def _gather_call(feats, ctab, srcg, dstg, epad, nch, interpret=False):
    cpt = nch * CW
    f32 = jnp.float32

    @functools.partial(
        pl.kernel,
        out_type=[
            jax.ShapeDtypeStruct((epad, F), f32),
            jax.ShapeDtypeStruct((epad, F), f32),
            jax.ShapeDtypeStruct((epad, F), f32),
            jax.ShapeDtypeStruct((epad, F), f32),
        ],
        mesh=_mesh(),
        scratch_types=[
            pltpu.VMEM((nch, CW), jnp.int32),
            pltpu.VMEM((nch, CW), jnp.int32),
            pltpu.VMEM((2, CW, F), f32),
            pltpu.VMEM((2, CW, F), f32),
            pltpu.VMEM((2, CW, CP), f32),
            pltpu.VMEM((2, CW, CP), f32),
            pltpu.SemaphoreType.DMA,
            pltpu.SemaphoreType.DMA,
            pltpu.SemaphoreType.DMA,
            pltpu.SemaphoreType.DMA,
        ],
        compiler_params=pltpu.CompilerParams(use_tc_tiling_on_sc=False),
        interpret=interpret,
    )
    def k(feats_hbm, ctab_hbm, srcg_hbm, dstg_hbm, xi_hbm, xj_hbm, ci_hbm, cj_hbm,
          idxs_v, idxd_v, r_a, r_b, r_c, r_d, gs0, gs1, ws0, ws1):
        c = lax.axis_index("c")
        s = lax.axis_index("s")
        base = (c * NS + s) * cpt
        gs = (gs0, gs1)
        ws = (ws0, ws1)
        pltpu.sync_copy(srcg_hbm.at[c, s], idxs_v)
        pltpu.sync_copy(dstg_hbm.at[c, s], idxd_v)

        def fire_gather(j, b):
            pltpu.async_copy(feats_hbm.at[idxd_v.at[j]], r_a.at[b], gs[b])
            pltpu.async_copy(feats_hbm.at[idxs_v.at[j]], r_b.at[b], gs[b])
            pltpu.async_copy(ctab_hbm.at[idxd_v.at[j]], r_c.at[b], gs[b])
            pltpu.async_copy(ctab_hbm.at[idxs_v.at[j]], r_d.at[b], gs[b])

        def wait_gather(b):
            pltpu.make_async_copy(feats_hbm.at[idxd_v.at[0]], r_a.at[b], gs[b]).wait()
            pltpu.make_async_copy(feats_hbm.at[idxs_v.at[0]], r_b.at[b], gs[b]).wait()
            pltpu.make_async_copy(ctab_hbm.at[idxd_v.at[0]], r_c.at[b], gs[b]).wait()
            pltpu.make_async_copy(ctab_hbm.at[idxs_v.at[0]], r_d.at[b], gs[b]).wait()

        def fire_writes(j, b):
            off = base + j * CW
            pltpu.async_copy(r_a.at[b], xi_hbm.at[pl.ds(off, CW)], ws[b])
            pltpu.async_copy(r_b.at[b], xj_hbm.at[pl.ds(off, CW)], ws[b])
            pltpu.async_copy(r_c.at[b], ci_hbm.at[pl.ds(off, CW), pl.ds(0, CP)], ws[b])
            pltpu.async_copy(r_d.at[b], cj_hbm.at[pl.ds(off, CW), pl.ds(0, CP)], ws[b])

        def wait_writes(b):
            pltpu.make_async_copy(r_a.at[b], xi_hbm.at[pl.ds(0, CW)], ws[b]).wait()
            pltpu.make_async_copy(r_b.at[b], xj_hbm.at[pl.ds(0, CW)], ws[b]).wait()
            pltpu.make_async_copy(r_c.at[b], ci_hbm.at[pl.ds(0, CW), pl.ds(0, CP)], ws[b]).wait()
            pltpu.make_async_copy(r_d.at[b], cj_hbm.at[pl.ds(0, CW), pl.ds(0, CP)], ws[b]).wait()

        def body(g, carry):
            j0 = 2 * g

            @pl.when(g > 0)
            def _():
                wait_writes(0)

            fire_gather(j0, 0)
            wait_gather(0)
            fire_writes(j0, 0)

            @pl.when(g > 0)
            def _():
                wait_writes(1)

            fire_gather(j0 + 1, 1)
            wait_gather(1)
            fire_writes(j0 + 1, 1)
            return carry

        lax.fori_loop(0, nch // 2, body, 0)
        wait_writes(0)
        wait_writes(1)

    return k(feats, ctab, srcg, dstg)


def _scatter_call(eout, dsts, zer, n, nd, epad, nch, interpret=False):
    cpt = nch * CW
    zr = nd // NS
    npr = n // NS
    f32 = jnp.float32

    @functools.partial(
        pl.kernel,
        out_type=jax.ShapeDtypeStruct((NC, n, F), f32),
        mesh=_mesh(),
        scratch_types=[
            pltpu.VMEM((nch, CW), jnp.int32),
            pltpu.VMEM((max(CW, npr), 32), f32),
            pltpu.VMEM_SHARED((nd, 32), f32),
            pltpu.SemaphoreType.DMA,
        ],
        compiler_params=pltpu.CompilerParams(use_tc_tiling_on_sc=False),
        interpret=interpret,
    )
    def k(eout_hbm, dsts_hbm, zer_hbm, out_hbm, idx_v, rows_v, acc_sh, sem):
        c = lax.axis_index("c")
        s = lax.axis_index("s")
        base = (c * NS + s) * cpt
        pltpu.sync_copy(dsts_hbm.at[c, s], idx_v)
        pltpu.sync_copy(zer_hbm.at[pl.ds(s * zr, zr)], acc_sh.at[pl.ds(s * zr, zr)])
        plsc.subcore_barrier()

        def body(j, carry):
            cp = pltpu.async_copy(
                eout_hbm.at[pl.ds(base + j * CW, CW), pl.ds(0, 32)],
                rows_v.at[pl.ds(0, CW)], sem,
            )
            cp.wait()
            pltpu.sync_copy(rows_v.at[pl.ds(0, CW)], acc_sh.at[idx_v.at[j]], add=True)
            return carry

        lax.fori_loop(0, nch, body, 0)
        plsc.subcore_barrier()
        pltpu.sync_copy(acc_sh.at[pl.ds(s * npr, npr)], rows_v.at[pl.ds(0, npr)])
        pltpu.sync_copy(rows_v.at[pl.ds(0, npr)],
                        out_hbm.at[c, pl.ds(s * npr, npr), pl.ds(0, 32)])

    return k(eout, dsts, zer)


def _edge_body(xi, xj, ci, cj, ea, w1ij, w1a, w1d, b1, w2, b2, wc1, bc1, wc2,
               bc2, out):
    rel = cj[:, :CP] - ci[:, :CP]
    rd = jnp.sum(rel * rel, axis=1, keepdims=True)
    xij = jnp.concatenate([xi[...], xj[...]], axis=1).astype(jnp.bfloat16)
    h = jnp.dot(xij, w1ij[...], preferred_element_type=jnp.float32)
    h = h + jnp.dot(ea[...], w1a[...], preferred_element_type=jnp.float32)
    h = h + rd * w1d[...] + b1[...]
    h = h * jax.nn.sigmoid(h)
    m = jnp.dot(h.astype(jnp.bfloat16), w2[...],
                preferred_element_type=jnp.float32) + b2[...]
    m = m * jax.nn.sigmoid(m)
    q = jnp.dot(m, wc1[...], preferred_element_type=jnp.float32) + bc1[...]
    q = q * jax.nn.sigmoid(q)
    cw = jnp.sum(q * wc2[...], axis=1, keepdims=True) + bc2[0, 0]
    out[:, :32] = jnp.concatenate([m, cw * rel], axis=1)


def _edge_call(xi, xj, ci, cj, ea, w1ij, w1a, w1d, b1, w2, b2, wc1, bc1, wc2,
               bc2, epad, be, interpret=False):
    grid = (epad // be,)
    row = lambda i: (i, 0)
    full = lambda i: (0, 0)
    return pl.pallas_call(
        _edge_body,
        grid=grid,
        in_specs=[
            pl.BlockSpec((be, F), row),
            pl.BlockSpec((be, F), row),
            pl.BlockSpec((be, F), row),
            pl.BlockSpec((be, F), row),
            pl.BlockSpec((be, 16), row),
            pl.BlockSpec((2 * F, H2), full),
            pl.BlockSpec((16, H2), full),
            pl.BlockSpec((1, H2), full),
            pl.BlockSpec((1, H2), full),
            pl.BlockSpec((H2, 16), full),
            pl.BlockSpec((1, 16), full),
            pl.BlockSpec((16, 64), full),
            pl.BlockSpec((1, 64), full),
            pl.BlockSpec((1, 64), full),
            pl.BlockSpec((1, 1), full),
        ],
        out_specs=pl.BlockSpec((be, F), row),
        out_shape=jax.ShapeDtypeStruct((epad, F), jnp.float32),
        interpret=interpret,
    )(xi, xj, ci, cj, ea, w1ij, w1a, w1d, b1, w2, b2, wc1, bc1, wc2, bc2)


def _node_call(feats, aggs, ctab, w1f, w1m, b1, w2, b2, n, bn, interpret=False):
    na = len(aggs)

    def body(*refs):
        feats_r = refs[0]
        agg_rs = refs[1:1 + na]
        ctab_r, w1f_r, w1m_r, b1_r, w2_r, b2_r = refs[1 + na:1 + na + 6]
        nfeats, nctab = refs[-2:]
        a = agg_rs[0][...]
        for r in agg_rs[1:]:
            a = a + r[...]
        mi = a[:, :16]
        mh = a[:, 16:32]
        nctab[...] = ctab_r[...] + mh
        t = jnp.dot(feats_r[...], w1f_r[...], preferred_element_type=jnp.float32)
        t = t + jnp.dot(mi, w1m_r[...], preferred_element_type=jnp.float32) + b1_r[...]
        t = t * jax.nn.sigmoid(t)
        nfeats[...] = feats_r[...] + jnp.dot(t, w2_r[...], preferred_element_type=jnp.float32) + b2_r[...]

    grid = (n // bn,)
    row = lambda i: (i, 0)
    full = lambda i: (0, 0)
    return pl.pallas_call(
        body,
        grid=grid,
        in_specs=[pl.BlockSpec((bn, F), row)]
        + [pl.BlockSpec((bn, F), row)] * na
        + [
            pl.BlockSpec((bn, CP), row),
            pl.BlockSpec((F, 256), full),
            pl.BlockSpec((16, 256), full),
            pl.BlockSpec((1, 256), full),
            pl.BlockSpec((256, F), full),
            pl.BlockSpec((1, F), full),
        ],
        out_specs=[
            pl.BlockSpec((bn, F), row),
            pl.BlockSpec((bn, CP), row),
        ],
        out_shape=[
            jax.ShapeDtypeStruct((n, F), jnp.float32),
            jax.ShapeDtypeStruct((n, CP), jnp.float32),
        ],
        interpret=interpret,
    )(feats, *aggs, ctab, w1f, w1m, b1, w2, b2)


def _pad2(w, rows, cols):
    return jnp.pad(w, ((0, rows - w.shape[0]), (0, cols - w.shape[1])))


def kernel(x, edge_index, batch, edge_attr, We1, be1, We2, be2, Wc1, bc1, Wc2,
           bc2, Wn1, bn1, Wn2, bn2):
    n = x.shape[0]
    e = edge_index.shape[1]
    n_layers = We1.shape[0]
    nch = (e + NW * CW - 1) // (NW * CW)
    nch = ((nch + 3) // 4) * 4
    epad = nch * NW * CW
    if nch >= 40 and (nch - 8) % 6 == 0:
        splits = [8] + [(nch - 8) // 3] * 3
    else:
        splits = [nch // 2, nch // 2]
    nd = ((n + 1 + NS - 1) // NS) * NS
    bn = 1000 if n % 1000 == 0 else n
    be = 2048

    src = edge_index[0]
    dst = edge_index[1]
    pad = epad - e
    spread = jnp.arange(pad, dtype=jnp.int32) % n
    srcg = jnp.concatenate([src, spread]).reshape(NC, NS, nch, CW)
    dstg = jnp.concatenate([dst, spread]).reshape(NC, NS, nch, CW)
    dsts = jnp.pad(dst, (0, pad), constant_values=n).reshape(NC, NS, nch, CW)
    eap = jnp.pad(edge_attr, ((0, pad), (0, 0))).reshape(NW, nch, CW, 16)
    zer = jnp.zeros((nd, 32), jnp.float32)
    offs = [sum(splits[:i]) for i in range(len(splits))]
    srcg_h = [srcg[:, :, o:o + s, :] for o, s in zip(offs, splits)]
    dstg_h = [dstg[:, :, o:o + s, :] for o, s in zip(offs, splits)]
    dsts_h = [dsts[:, :, o:o + s, :] for o, s in zip(offs, splits)]
    eap_h = [eap[:, o:o + s].reshape(s * NW * CW, 16) for o, s in zip(offs, splits)]

    feats = x[:, 3:]
    ctab = jnp.pad(x[:, :3], ((0, 0), (0, CP - 3)))

    for l in range(n_layers):
        w1 = We1[l]
        w1ij = _pad2(w1[:2 * F], 2 * F, H2).astype(jnp.bfloat16)
        w1a = _pad2(w1[2 * F:2 * F + 16], 16, H2)
        w1d = _pad2(w1[2 * F + 16:], 1, H2)
        b1 = _pad2(be1[l][None, :], 1, H2)
        w2 = _pad2(We2[l], H2, 16).astype(jnp.bfloat16)
        b2 = be2[l][None, :]
        wc1 = Wc1[l]
        bc1l = bc1[l][None, :]
        wc2 = Wc2[l].reshape(1, 64)
        bc2l = bc2[l].reshape(1, 1)
        w1f = Wn1[l][:F]
        w1m = Wn1[l][F:]
        bn1l = bn1[l][None, :]
        wn2 = Wn2[l]
        bn2l = bn2[l][None, :]

        aggs = []
        for h, s in enumerate(splits):
            epad_s = s * NW * CW
            xi, xj, ci, cj = _gather_call(feats, ctab, srcg_h[h], dstg_h[h],
                                          epad_s, s)
            eout = _edge_call(xi, xj, ci, cj, eap_h[h], w1ij, w1a, w1d, b1,
                              w2, b2, wc1, bc1l, wc2, bc2l, epad_s, be)
            ah = _scatter_call(eout, dsts_h[h], zer, n, nd, epad_s, s)
            aggs.extend([ah[0], ah[1]])
        feats, ctab = _node_call(feats, aggs, ctab, w1f, w1m, bn1l, wn2, bn2l,
                                 n, bn)

    return jnp.concatenate([ctab[:, :3], feats], axis=1)
